# baseline (speedup 1.0000x reference)
import functools

import jax
import jax.numpy as jnp
from jax import lax
from jax.experimental import pallas as pl
from jax.experimental.pallas import tpu as pltpu
from jax.experimental.pallas import tpu_sc as plsc

B = 128
V = 100000
NC = 2
NS = 16
NW = NC * NS
ROWS_PER_W = B // NW
L = 16
CH = 10000
NCH = V // CH
INNER = CH // L
I32MAX = 2147483647


def _recip(x):
    xi = plsc.bitcast(x, jnp.int32)
    y = plsc.bitcast(jnp.int32(0x7EF311C3) - xi, jnp.float32)
    for _ in range(4):
        y = y * (2.0 - x * y)
    return y


def _sampler_body(logits_hbm, temps_hbm, noise_hbm, out_hbm,
                  lrow, nb0, nb1, tbuf, obuf, sem0, sem1):
    c = lax.axis_index("c")
    s = lax.axis_index("s")
    wid = s * NC + c
    lane = lax.iota(jnp.int32, L)

    pltpu.sync_copy(temps_hbm, tbuf.at[pl.ds(0, B)])
    t_vec = tbuf[pl.ds(wid * ROWS_PER_W, L)]

    acc = jnp.zeros((L,), jnp.int32)
    for r in range(ROWS_PER_W):
        row = wid * ROWS_PER_W + r

        pltpu.async_copy(noise_hbm.at[row, pl.ds(0, CH)], nb0, sem0)
        pltpu.async_copy(noise_hbm.at[row, pl.ds(CH, CH)], nb1, sem1)

        pltpu.sync_copy(logits_hbm.at[row], lrow)

        def p1(i, carry):
            maxv, gidx, idxv = carry
            l = lrow[pl.ds(i * L, L)]
            upd = l > maxv
            gidx = jnp.where(upd, idxv, gidx)
            maxv = jnp.maximum(maxv, l)
            return maxv, gidx, idxv + L

        maxv, gidx, _ = lax.fori_loop(
            0, V // L, p1,
            (jnp.full((L,), -jnp.inf, jnp.float32), jnp.zeros((L,), jnp.int32),
             lane),
            unroll=10)
        m_l = jnp.max(maxv)
        g_tok = jnp.min(jnp.where(maxv == m_l, gidx, I32MAX))

        t_r = jnp.max(jnp.where(lane == r, t_vec, -jnp.inf))
        tpos = t_r > 0.0
        safe_t = jnp.where(tpos, jnp.full((L,), t_r, jnp.float32),
                           jnp.full((L,), 1.0, jnp.float32))
        inv_t = _recip(safe_t)
        m2 = jnp.full((L,), m_l, jnp.float32) * inv_t

        def p2_chunk(nb, cidx, carry):
            def body(j, carry):
                b_e, b_n, bidx, idxv = carry
                l = lrow[pl.ds(cidx * CH + j * L, L)]
                n = nb[pl.ds(j * L, L)]
                e = jnp.exp(l * inv_t - m2)
                ncl = jnp.maximum(n, 1e-10)
                upd = e * b_n > b_e * ncl
                b_e = jnp.where(upd, e, b_e)
                b_n = jnp.where(upd, ncl, b_n)
                bidx = jnp.where(upd, idxv, bidx)
                return b_e, b_n, bidx, idxv + L
            return lax.fori_loop(0, INNER, body, carry, unroll=5)

        def p2_outer(k, carry):
            c0 = 2 * k
            pltpu.make_async_copy(noise_hbm.at[row, pl.ds(0, CH)], nb0,
                                  sem0).wait()
            carry = p2_chunk(nb0, c0, carry)

            @pl.when(c0 + 2 < NCH)
            def _():
                pltpu.async_copy(
                    noise_hbm.at[row, pl.ds((c0 + 2) * CH, CH)], nb0, sem0)

            pltpu.make_async_copy(noise_hbm.at[row, pl.ds(0, CH)], nb1,
                                  sem1).wait()
            carry = p2_chunk(nb1, c0 + 1, carry)

            @pl.when(c0 + 3 < NCH)
            def _():
                pltpu.async_copy(
                    noise_hbm.at[row, pl.ds((c0 + 3) * CH, CH)], nb1, sem1)

            return carry

        init = (jnp.full((L,), -1.0, jnp.float32),
                jnp.ones((L,), jnp.float32),
                jnp.zeros((L,), jnp.int32),
                lane)
        b_e, b_n, bidx, _ = lax.fori_loop(0, NCH // 2, p2_outer, init)

        key = b_e * _recip(b_n)
        m_k = jnp.max(key)
        s_tok = jnp.min(jnp.where(key == m_k, bidx, I32MAX))

        tok = jnp.where(tpos, s_tok, g_tok)
        acc = jnp.where(lane == r, tok, acc)

    obuf[...] = acc
    pltpu.sync_copy(obuf, out_hbm.at[wid])


@jax.jit
def _sampler(logits, temperatures, exp_noise):
    mesh = plsc.VectorSubcoreMesh(core_axis_name="c", subcore_axis_name="s",
                                  num_cores=NC, num_subcores=NS)
    f = pl.kernel(
        _sampler_body,
        out_type=jax.ShapeDtypeStruct((NW, L), jnp.int32),
        mesh=mesh,
        scratch_types=[
            pltpu.VMEM((V,), jnp.float32),
            pltpu.VMEM((CH,), jnp.float32),
            pltpu.VMEM((CH,), jnp.float32),
            pltpu.VMEM((B + 2 * L,), jnp.float32),
            pltpu.VMEM((L,), jnp.int32),
            pltpu.SemaphoreType.DMA,
            pltpu.SemaphoreType.DMA,
        ],
        compiler_params=pltpu.CompilerParams(use_tc_tiling_on_sc=False,
                                             needs_layout_passes=False),
    )
    return f(logits, temperatures, exp_noise)


def kernel(logits, temperatures, exp_noise):
    out2d = _sampler(logits, temperatures, exp_noise)
    return out2d[:, :ROWS_PER_W].reshape(B)


# speedup vs baseline: 1.5116x; 1.5116x over previous
import functools

import jax
import jax.numpy as jnp
from jax import lax
from jax.experimental import pallas as pl
from jax.experimental.pallas import tpu as pltpu
from jax.experimental.pallas import tpu_sc as plsc

B = 128
V = 100000
NC = 2
NS = 16
NW = NC * NS
ROWS_PER_W = B // NW
L = 16
CH = 9984
NCH = 10
TAIL = V - NCH * CH
INNER = CH // L
I32MAX = 2147483647


def _recip(x):
    xi = plsc.bitcast(x, jnp.int32)
    y = plsc.bitcast(jnp.int32(0x7EF311C3) - xi, jnp.float32)
    for _ in range(4):
        y = y * (2.0 - x * y)
    return y


def _sampler_body(logits_hbm, temps_hbm, noise_hbm, out_hbm,
                  lrow, nb0, nb1, ntail8, tbuf, obuf, sem0, sem1):
    c = lax.axis_index("c")
    s = lax.axis_index("s")
    wid = s * NC + c
    lane = lax.iota(jnp.int32, L)

    pltpu.sync_copy(temps_hbm, tbuf.at[pl.ds(0, B)])
    t_vec = tbuf[pl.ds(wid * ROWS_PER_W, L)]

    pltpu.sync_copy(
        noise_hbm.at[pl.ds(8 * (wid // 2), 8), pl.ds(NCH * CH, TAIL)], ntail8)

    acc = jnp.zeros((L,), jnp.int32)
    for r in range(ROWS_PER_W):
        row = wid * ROWS_PER_W + r

        pltpu.async_copy(noise_hbm.at[row, pl.ds(0, CH)], nb0, sem0)
        pltpu.async_copy(noise_hbm.at[row, pl.ds(CH, CH)], nb1, sem1)

        pltpu.sync_copy(logits_hbm.at[row], lrow)

        def p1(i, carry):
            maxv, gidx, idxv = carry
            l = lrow[pl.ds(i * L, L)]
            upd = l > maxv
            gidx = jnp.where(upd, idxv, gidx)
            maxv = jnp.maximum(maxv, l)
            return maxv, gidx, idxv + L

        maxv, gidx, _ = lax.fori_loop(
            0, V // L, p1,
            (jnp.full((L,), -jnp.inf, jnp.float32), jnp.zeros((L,), jnp.int32),
             lane),
            unroll=10)
        m_l = jnp.max(maxv)
        g_tok = jnp.min(jnp.where(maxv == m_l, gidx, I32MAX))

        t_r = jnp.max(jnp.where(lane == r, t_vec, -jnp.inf))
        tpos = t_r > 0.0
        safe_t = jnp.where(tpos, jnp.full((L,), t_r, jnp.float32),
                           jnp.full((L,), 1.0, jnp.float32))
        inv_t = _recip(safe_t)
        m2 = jnp.full((L,), m_l, jnp.float32) * inv_t

        def p2_step(l, n, idxv, carry):
            b_e, b_n, bidx = carry
            e = jnp.exp(l * inv_t - m2)
            ncl = jnp.maximum(n, 1e-10)
            upd = e * b_n > b_e * ncl
            b_e = jnp.where(upd, e, b_e)
            b_n = jnp.where(upd, ncl, b_n)
            bidx = jnp.where(upd, idxv, bidx)
            return b_e, b_n, bidx

        def p2_chunk(nb, cidx, carry):
            def body(j, carry):
                b_e, b_n, bidx, idxv = carry
                l = lrow[pl.ds(cidx * CH + j * L, L)]
                n = nb[pl.ds(j * L, L)]
                b_e, b_n, bidx = p2_step(l, n, idxv, (b_e, b_n, bidx))
                return b_e, b_n, bidx, idxv + L
            return lax.fori_loop(0, INNER, body, carry, unroll=8)

        def p2_outer(k, carry):
            c0 = 2 * k
            pltpu.make_async_copy(noise_hbm.at[row, pl.ds(0, CH)], nb0,
                                  sem0).wait()
            carry = p2_chunk(nb0, c0, carry)

            @pl.when(c0 + 2 < NCH)
            def _():
                pltpu.async_copy(
                    noise_hbm.at[row, pl.ds((c0 + 2) * CH, CH)], nb0, sem0)

            pltpu.make_async_copy(noise_hbm.at[row, pl.ds(0, CH)], nb1,
                                  sem1).wait()
            carry = p2_chunk(nb1, c0 + 1, carry)

            @pl.when(c0 + 3 < NCH)
            def _():
                pltpu.async_copy(
                    noise_hbm.at[row, pl.ds((c0 + 3) * CH, CH)], nb1, sem1)

            return carry

        init = (jnp.full((L,), -1.0, jnp.float32),
                jnp.ones((L,), jnp.float32),
                jnp.zeros((L,), jnp.int32),
                lane)
        b_e, b_n, bidx, idxv = lax.fori_loop(0, NCH // 2, p2_outer, init)

        carry = (b_e, b_n, bidx)
        for j in range(TAIL // L):
            l = lrow[pl.ds(NCH * CH + j * L, L)]
            n = ntail8[4 * (wid % 2) + r, pl.ds(j * L, L)]
            carry = p2_step(l, n, idxv + j * L, carry)
        b_e, b_n, bidx = carry

        key = b_e * _recip(b_n)
        m_k = jnp.max(key)
        s_tok = jnp.min(jnp.where(key == m_k, bidx, I32MAX))

        tok = jnp.where(tpos, s_tok, g_tok)
        acc = jnp.where(lane == r, tok, acc)

    obuf[...] = acc
    pltpu.sync_copy(obuf, out_hbm.at[wid])


@jax.jit
def _sampler(logits, temperatures, exp_noise):
    mesh = plsc.VectorSubcoreMesh(core_axis_name="c", subcore_axis_name="s",
                                  num_cores=NC, num_subcores=NS)
    f = pl.kernel(
        _sampler_body,
        out_type=jax.ShapeDtypeStruct((NW, L), jnp.int32),
        mesh=mesh,
        scratch_types=[
            pltpu.VMEM((V,), jnp.float32),
            pltpu.VMEM((CH,), jnp.float32),
            pltpu.VMEM((CH,), jnp.float32),
            pltpu.VMEM((8, TAIL), jnp.float32),
            pltpu.VMEM((B + 2 * L,), jnp.float32),
            pltpu.VMEM((L,), jnp.int32),
            pltpu.SemaphoreType.DMA,
            pltpu.SemaphoreType.DMA,
        ],
        compiler_params=pltpu.CompilerParams(needs_layout_passes=False),
    )
    return f(logits, temperatures, exp_noise)


def kernel(logits, temperatures, exp_noise):
    out2d = _sampler(logits, temperatures, exp_noise)
    return out2d[:, :ROWS_PER_W].reshape(B)


# speedup vs baseline: 1.5306x; 1.0125x over previous
import functools

import jax
import jax.numpy as jnp
from jax import lax
from jax.experimental import pallas as pl
from jax.experimental.pallas import tpu as pltpu
from jax.experimental.pallas import tpu_sc as plsc

B = 128
V = 100000
R_SC = 32
NC = 2
NS = 16
NW = NC * NS
RPW = R_SC // NW
L = 16
CH = 9984
NCH = 10
TAIL = V - NCH * CH
INNER = CH // L
I32MAX = 2147483647

WCH = 12800
NJ = -(-V // WCH)


def _recip(x):
    xi = plsc.bitcast(x, jnp.int32)
    y = plsc.bitcast(jnp.int32(0x7EF311C3) - xi, jnp.float32)
    for _ in range(4):
        y = y * (2.0 - x * y)
    return y


def _sc_body(logits_hbm, temps_hbm, noise_hbm, out_hbm,
             lrow, nb0, nb1, ntail8, tbuf, obuf, sem0, sem1):
    c = lax.axis_index("c")
    s = lax.axis_index("s")
    wid = s * NC + c
    lane = lax.iota(jnp.int32, L)

    pltpu.sync_copy(temps_hbm, tbuf.at[pl.ds(0, B)])
    t_vec = tbuf[pl.ds(wid * RPW, L)]

    grp = 8 * ((wid * RPW) // 8)
    pltpu.sync_copy(
        noise_hbm.at[pl.ds(grp, 8), pl.ds(NCH * CH, TAIL)], ntail8)

    acc = jnp.zeros((L,), jnp.int32)
    for r in range(RPW):
        row = wid * RPW + r

        pltpu.async_copy(noise_hbm.at[row, pl.ds(0, CH)], nb0, sem0)
        pltpu.async_copy(noise_hbm.at[row, pl.ds(CH, CH)], nb1, sem1)

        pltpu.sync_copy(logits_hbm.at[row], lrow)

        def p1(i, carry):
            maxv, gidx, idxv = carry
            l = lrow[pl.ds(i * L, L)]
            upd = l > maxv
            gidx = jnp.where(upd, idxv, gidx)
            maxv = jnp.maximum(maxv, l)
            return maxv, gidx, idxv + L

        maxv, gidx, _ = lax.fori_loop(
            0, V // L, p1,
            (jnp.full((L,), -jnp.inf, jnp.float32), jnp.zeros((L,), jnp.int32),
             lane),
            unroll=10)
        m_l = jnp.max(maxv)
        g_tok = jnp.min(jnp.where(maxv == m_l, gidx, I32MAX))

        t_r = jnp.max(jnp.where(lane == r, t_vec, -jnp.inf))
        tpos = t_r > 0.0
        safe_t = jnp.where(tpos, jnp.full((L,), t_r, jnp.float32),
                           jnp.full((L,), 1.0, jnp.float32))
        inv_t = _recip(safe_t)
        m2 = jnp.full((L,), m_l, jnp.float32) * inv_t

        def p2_step(l, n, idxv, carry):
            b_e, b_n, bidx = carry
            e = jnp.exp(l * inv_t - m2)
            ncl = jnp.maximum(n, 1e-10)
            upd = e * b_n > b_e * ncl
            b_e = jnp.where(upd, e, b_e)
            b_n = jnp.where(upd, ncl, b_n)
            bidx = jnp.where(upd, idxv, bidx)
            return b_e, b_n, bidx

        def p2_chunk(nb, cidx, carry):
            def body(j, carry):
                b_e, b_n, bidx, idxv = carry
                l = lrow[pl.ds(cidx * CH + j * L, L)]
                n = nb[pl.ds(j * L, L)]
                b_e, b_n, bidx = p2_step(l, n, idxv, (b_e, b_n, bidx))
                return b_e, b_n, bidx, idxv + L
            return lax.fori_loop(0, INNER, body, carry, unroll=8)

        def p2_outer(k, carry):
            c0 = 2 * k
            pltpu.make_async_copy(noise_hbm.at[row, pl.ds(0, CH)], nb0,
                                  sem0).wait()
            carry = p2_chunk(nb0, c0, carry)

            @pl.when(c0 + 2 < NCH)
            def _():
                pltpu.async_copy(
                    noise_hbm.at[row, pl.ds((c0 + 2) * CH, CH)], nb0, sem0)

            pltpu.make_async_copy(noise_hbm.at[row, pl.ds(0, CH)], nb1,
                                  sem1).wait()
            carry = p2_chunk(nb1, c0 + 1, carry)

            @pl.when(c0 + 3 < NCH)
            def _():
                pltpu.async_copy(
                    noise_hbm.at[row, pl.ds((c0 + 3) * CH, CH)], nb1, sem1)

            return carry

        init = (jnp.full((L,), -1.0, jnp.float32),
                jnp.ones((L,), jnp.float32),
                jnp.zeros((L,), jnp.int32),
                lane)
        b_e, b_n, bidx, idxv = lax.fori_loop(0, NCH // 2, p2_outer, init)

        carry = (b_e, b_n, bidx)
        rmod = row - grp
        for j in range(TAIL // L):
            l = lrow[pl.ds(NCH * CH + j * L, L)]
            n = ntail8[rmod, pl.ds(j * L, L)]
            carry = p2_step(l, n, idxv + j * L, carry)
        b_e, b_n, bidx = carry

        key = b_e * _recip(b_n)
        m_k = jnp.max(key)
        s_tok = jnp.min(jnp.where(key == m_k, bidx, I32MAX))

        tok = jnp.where(tpos, s_tok, g_tok)
        acc = jnp.where(lane == r, tok, acc)

    obuf[...] = acc
    pltpu.sync_copy(obuf, out_hbm.at[wid])


def _sc_sampler(logits, temperatures, exp_noise):
    mesh = plsc.VectorSubcoreMesh(core_axis_name="c", subcore_axis_name="s",
                                  num_cores=NC, num_subcores=NS)
    f = pl.kernel(
        _sc_body,
        out_type=jax.ShapeDtypeStruct((NW, L), jnp.int32),
        mesh=mesh,
        scratch_types=[
            pltpu.VMEM((V,), jnp.float32),
            pltpu.VMEM((CH,), jnp.float32),
            pltpu.VMEM((CH,), jnp.float32),
            pltpu.VMEM((8, TAIL), jnp.float32),
            pltpu.VMEM((B + 2 * L,), jnp.float32),
            pltpu.VMEM((L,), jnp.int32),
            pltpu.SemaphoreType.DMA,
            pltpu.SemaphoreType.DMA,
        ],
        compiler_params=pltpu.CompilerParams(needs_layout_passes=False),
    )
    out2d = f(logits, temperatures, exp_noise)
    return out2d[:, :RPW].reshape(R_SC)


def _tc_body(lb, tb, nb, out, st_m, st_e, st_n, st_i, st_gm, st_gi):
    j = pl.program_id(1)

    t8 = tb[0, 0, :].reshape(8, 1)
    tpos = t8 > 0.0
    invt = 1.0 / jnp.where(tpos, t8, 1.0)

    idx = j * WCH + lax.broadcasted_iota(jnp.int32, (8, WCH), 1)
    mask = idx < V
    lblk = jnp.where(mask, lb[0], -jnp.inf)
    nblk = jnp.where(mask, nb[0], jnp.inf)

    gm = jnp.max(lblk, axis=1, keepdims=True)
    gi = jnp.min(jnp.where(lblk == gm, idx, I32MAX), axis=1)

    scaled = lblk * invt
    mc = jnp.max(scaled, axis=1, keepdims=True)
    e = jnp.exp(scaled - mc)
    ncl = jnp.maximum(nblk, 1e-10)
    r = e / ncl
    rm = jnp.max(r, axis=1, keepdims=True)
    ri = jnp.min(jnp.where(r == rm, idx, I32MAX), axis=1, keepdims=True)
    wsel = idx == ri
    we = jnp.max(jnp.where(wsel, e, -1.0), axis=1)
    wn = jnp.max(jnp.where(wsel, ncl, -1.0), axis=1)

    st_m[j, :] = mc[:, 0]
    st_e[j, :] = we
    st_n[j, :] = wn
    st_i[j, :] = ri[:, 0]
    st_gm[j, :] = gm[:, 0]
    st_gi[j, :] = gi

    @pl.when(j == NJ - 1)
    def _():
        m_all = st_m[...]
        mg = jnp.max(m_all, axis=0, keepdims=True)
        key = (st_e[...] * jnp.exp(m_all - mg)) / st_n[...]
        km = jnp.max(key, axis=0, keepdims=True)
        stok = jnp.min(jnp.where(key == km, st_i[...], I32MAX), axis=0)

        gm_all = st_gm[...]
        g = jnp.max(gm_all, axis=0, keepdims=True)
        gtok = jnp.min(jnp.where(gm_all == g, st_gi[...], I32MAX), axis=0)

        out[0, 0, :] = jnp.where(tpos[:, 0], stok, gtok)


def _tc_sampler(logits, temperatures, exp_noise):
    NTC = B - R_SC
    NRB = NTC // 8
    f = pl.pallas_call(
        _tc_body,
        grid=(NRB, NJ),
        in_specs=[
            pl.BlockSpec((1, 8, WCH), lambda i, j: (0, R_SC // 8 + i, j)),
            pl.BlockSpec((1, 1, 8), lambda i, j: (R_SC // 8 + i, 0, 0)),
            pl.BlockSpec((1, 8, WCH), lambda i, j: (0, R_SC // 8 + i, j)),
        ],
        out_specs=pl.BlockSpec((1, 1, 8), lambda i, j: (i, 0, 0)),
        out_shape=jax.ShapeDtypeStruct((NRB, 1, 8), jnp.int32),
        scratch_shapes=[
            pltpu.VMEM((NJ, 8), jnp.float32),
            pltpu.VMEM((NJ, 8), jnp.float32),
            pltpu.VMEM((NJ, 8), jnp.float32),
            pltpu.VMEM((NJ, 8), jnp.int32),
            pltpu.VMEM((NJ, 8), jnp.float32),
            pltpu.VMEM((NJ, 8), jnp.int32),
        ],
    )
    out = f(logits.reshape(1, B, V), temperatures.reshape(B // 8, 1, 8),
            exp_noise.reshape(1, B, V))
    return out.reshape(NTC)


@jax.jit
def _sampler(logits, temperatures, exp_noise):
    toks_sc = _sc_sampler(logits, temperatures, exp_noise)
    toks_tc = _tc_sampler(logits, temperatures, exp_noise)
    return jnp.concatenate([toks_sc, toks_tc])


def kernel(logits, temperatures, exp_noise):
    return _sampler(logits, temperatures, exp_noise)


# speedup vs baseline: 1.8527x; 1.2104x over previous
import functools

import jax
import jax.numpy as jnp
from jax import lax
from jax.experimental import pallas as pl
from jax.experimental.pallas import tpu as pltpu
from jax.experimental.pallas import tpu_sc as plsc

B = 128
V = 100000
R_SC = 64
NC = 2
NS = 16
NW = NC * NS
RPW = R_SC // NW
L = 16
CH = 9984
NCH = 10
TAIL = V - NCH * CH
INNER = CH // L
I32MAX = 2147483647

WCH = 12800
NJ = -(-V // WCH)


def _recip(x):
    xi = plsc.bitcast(x, jnp.int32)
    y = plsc.bitcast(jnp.int32(0x7EF311C3) - xi, jnp.float32)
    for _ in range(4):
        y = y * (2.0 - x * y)
    return y


def _sc_body(logits_hbm, temps_hbm, noise_hbm, out_hbm,
             lrow, nb0, nb1, ntail8, ltail8, tbuf, obuf, sem0, sem1, seml):
    c = lax.axis_index("c")
    s = lax.axis_index("s")
    wid = s * NC + c
    lane = lax.iota(jnp.int32, L)

    pltpu.sync_copy(temps_hbm, tbuf.at[pl.ds(0, B)])
    t_vec = tbuf[pl.ds(wid * RPW, L)]

    grp = 8 * ((wid * RPW) // 8)
    pltpu.sync_copy(
        noise_hbm.at[pl.ds(grp, 8), pl.ds(NCH * CH, TAIL)], ntail8)
    pltpu.sync_copy(
        logits_hbm.at[pl.ds(grp, 8), pl.ds(NCH * CH, TAIL)], ltail8)

    acc = jnp.zeros((L,), jnp.int32)
    for r in range(RPW):
        row = wid * RPW + r

        pltpu.async_copy(noise_hbm.at[row, pl.ds(0, CH)], nb0, sem0)
        pltpu.async_copy(noise_hbm.at[row, pl.ds(CH, CH)], nb1, sem1)

        def fire(k, _):
            pltpu.async_copy(logits_hbm.at[row, pl.ds(k * CH, CH)],
                             lrow.at[pl.ds(k * CH, CH)], seml)
            return 0
        lax.fori_loop(0, NCH, fire, 0)

        def p1_chunk(k, carry):
            pltpu.make_async_copy(logits_hbm.at[row, pl.ds(0, CH)],
                                  lrow.at[pl.ds(0, CH)], seml).wait()
            base = k * CH

            def p1_inner(i, carry):
                maxv, gidx, idxv = carry
                l = lrow[pl.ds(base + i * L, L)]
                upd = l > maxv
                gidx = jnp.where(upd, idxv, gidx)
                maxv = jnp.maximum(maxv, l)
                return maxv, gidx, idxv + L

            return lax.fori_loop(0, INNER, p1_inner, carry, unroll=8)

        carry1 = (jnp.full((L,), -jnp.inf, jnp.float32),
                  jnp.zeros((L,), jnp.int32), lane)
        maxv, gidx, idxv1 = lax.fori_loop(0, NCH, p1_chunk, carry1)
        rmod = row - grp
        for t in range(TAIL // L):
            l = ltail8[rmod, pl.ds(t * L, L)]
            upd = l > maxv
            gidx = jnp.where(upd, idxv1, gidx)
            maxv = jnp.maximum(maxv, l)
            idxv1 = idxv1 + L
        m_l = jnp.max(maxv)
        g_tok = jnp.min(jnp.where(maxv == m_l, gidx, I32MAX))

        t_r = jnp.max(jnp.where(lane == r, t_vec, -jnp.inf))
        tpos = t_r > 0.0
        safe_t = jnp.where(tpos, jnp.full((L,), t_r, jnp.float32),
                           jnp.full((L,), 1.0, jnp.float32))
        inv_t = _recip(safe_t)
        m2 = jnp.full((L,), m_l, jnp.float32) * inv_t

        def p2_step(l, n, idxv, carry):
            b_e, b_n, bidx = carry
            e = jnp.exp(l * inv_t - m2)
            ncl = jnp.maximum(n, 1e-10)
            upd = e * b_n > b_e * ncl
            b_e = jnp.where(upd, e, b_e)
            b_n = jnp.where(upd, ncl, b_n)
            bidx = jnp.where(upd, idxv, bidx)
            return b_e, b_n, bidx

        def p2_chunk(nb, cidx, carry):
            def body(j, carry):
                b_e, b_n, bidx, idxv = carry
                l = lrow[pl.ds(cidx * CH + j * L, L)]
                n = nb[pl.ds(j * L, L)]
                b_e, b_n, bidx = p2_step(l, n, idxv, (b_e, b_n, bidx))
                return b_e, b_n, bidx, idxv + L
            return lax.fori_loop(0, INNER, body, carry, unroll=8)

        def p2_outer(k, carry):
            c0 = 2 * k
            pltpu.make_async_copy(noise_hbm.at[row, pl.ds(0, CH)], nb0,
                                  sem0).wait()
            carry = p2_chunk(nb0, c0, carry)

            @pl.when(c0 + 2 < NCH)
            def _():
                pltpu.async_copy(
                    noise_hbm.at[row, pl.ds((c0 + 2) * CH, CH)], nb0, sem0)

            pltpu.make_async_copy(noise_hbm.at[row, pl.ds(0, CH)], nb1,
                                  sem1).wait()
            carry = p2_chunk(nb1, c0 + 1, carry)

            @pl.when(c0 + 3 < NCH)
            def _():
                pltpu.async_copy(
                    noise_hbm.at[row, pl.ds((c0 + 3) * CH, CH)], nb1, sem1)

            return carry

        init = (jnp.full((L,), -1.0, jnp.float32),
                jnp.ones((L,), jnp.float32),
                jnp.zeros((L,), jnp.int32),
                lane)
        b_e, b_n, bidx, idxv = lax.fori_loop(0, NCH // 2, p2_outer, init)

        carry = (b_e, b_n, bidx)
        rmod = row - grp
        for j in range(TAIL // L):
            l = lrow[pl.ds(NCH * CH + j * L, L)]
            n = ntail8[rmod, pl.ds(j * L, L)]
            carry = p2_step(l, n, idxv + j * L, carry)
        b_e, b_n, bidx = carry

        key = b_e * _recip(b_n)
        m_k = jnp.max(key)
        s_tok = jnp.min(jnp.where(key == m_k, bidx, I32MAX))

        tok = jnp.where(tpos, s_tok, g_tok)
        acc = jnp.where(lane == r, tok, acc)

    obuf[...] = acc
    pltpu.sync_copy(obuf, out_hbm.at[wid])


def _sc_sampler(logits, temperatures, exp_noise):
    mesh = plsc.VectorSubcoreMesh(core_axis_name="c", subcore_axis_name="s",
                                  num_cores=NC, num_subcores=NS)
    f = pl.kernel(
        _sc_body,
        out_type=jax.ShapeDtypeStruct((NW, L), jnp.int32),
        mesh=mesh,
        scratch_types=[
            pltpu.VMEM((V,), jnp.float32),
            pltpu.VMEM((CH,), jnp.float32),
            pltpu.VMEM((CH,), jnp.float32),
            pltpu.VMEM((8, TAIL), jnp.float32),
            pltpu.VMEM((8, TAIL), jnp.float32),
            pltpu.VMEM((B + 2 * L,), jnp.float32),
            pltpu.VMEM((L,), jnp.int32),
            pltpu.SemaphoreType.DMA,
            pltpu.SemaphoreType.DMA,
            pltpu.SemaphoreType.DMA,
        ],
        compiler_params=pltpu.CompilerParams(needs_layout_passes=False),
    )
    out2d = f(logits, temperatures, exp_noise)
    return out2d[:, :RPW].reshape(R_SC)


def _tc_body(lb, tb, nb, out, st_m, st_r, st_i, st_gm, st_gi):
    j = pl.program_id(1)

    t8 = tb[0, 0, :].reshape(8, 1)
    tpos = t8 > 0.0
    invt = 1.0 / jnp.where(tpos, t8, 1.0)

    idx = j * WCH + lax.broadcasted_iota(jnp.int32, (8, WCH), 1)

    def chunk_race(lblk, nblk):
        gm = jnp.max(lblk, axis=1, keepdims=True)

        @pl.when(jnp.any(~tpos))
        def _():
            st_gi[j, :] = jnp.min(jnp.where(lblk == gm, idx, I32MAX), axis=1)

        e = jnp.exp((lblk - gm) * invt)
        ncl = jnp.maximum(nblk, 1e-10)
        r = e / ncl
        rm = jnp.max(r, axis=1, keepdims=True)
        ri = jnp.min(jnp.where(r == rm, idx, I32MAX), axis=1)

        st_m[j, :] = (gm * invt)[:, 0]
        st_r[j, :] = rm[:, 0]
        st_i[j, :] = ri
        st_gm[j, :] = gm[:, 0]

    @pl.when(j < NJ - 1)
    def _():
        chunk_race(lb[...], nb[...])

    @pl.when(j == NJ - 1)
    def _():
        mask = idx < V
        chunk_race(jnp.where(mask, lb[...], -jnp.inf),
                   jnp.where(mask, nb[...], jnp.inf))

    @pl.when(j == NJ - 1)
    def _():
        m_all = st_m[...]
        mg = jnp.max(m_all, axis=0, keepdims=True)
        key = st_r[...] * jnp.exp(m_all - mg)
        km = jnp.max(key, axis=0, keepdims=True)
        stok = jnp.min(jnp.where(key == km, st_i[...], I32MAX), axis=0)

        gm_all = st_gm[...]
        g = jnp.max(gm_all, axis=0, keepdims=True)
        gtok = jnp.min(jnp.where(gm_all == g, st_gi[...], I32MAX), axis=0)

        out[0, 0, :] = jnp.where(tpos[:, 0], stok, gtok)


def _tc_sampler(logits, temperatures, exp_noise):
    NTC = B - R_SC
    NRB = NTC // 8
    f = pl.pallas_call(
        _tc_body,
        grid=(NRB, NJ),
        in_specs=[
            pl.BlockSpec((8, WCH), lambda i, j: (R_SC // 8 + i, j)),
            pl.BlockSpec((1, 1, 8), lambda i, j: (R_SC // 8 + i, 0, 0)),
            pl.BlockSpec((8, WCH), lambda i, j: (R_SC // 8 + i, j)),
        ],
        out_specs=pl.BlockSpec((1, 1, 8), lambda i, j: (i, 0, 0)),
        out_shape=jax.ShapeDtypeStruct((NRB, 1, 8), jnp.int32),
        scratch_shapes=[
            pltpu.VMEM((NJ, 8), jnp.float32),
            pltpu.VMEM((NJ, 8), jnp.float32),
            pltpu.VMEM((NJ, 8), jnp.int32),
            pltpu.VMEM((NJ, 8), jnp.float32),
            pltpu.VMEM((NJ, 8), jnp.int32),
        ],
    )
    out = f(logits, temperatures.reshape(B // 8, 1, 8), exp_noise)
    return out.reshape(NTC)


@jax.jit
def _sampler(logits, temperatures, exp_noise):
    toks_sc = _sc_sampler(logits, temperatures, exp_noise)
    toks_tc = _tc_sampler(logits, temperatures, exp_noise)
    return jnp.concatenate([toks_sc, toks_tc])


def kernel(logits, temperatures, exp_noise):
    return _sampler(logits, temperatures, exp_noise)


# speedup vs baseline: 3.9704x; 2.1431x over previous
import functools

import jax
import jax.numpy as jnp
from jax import lax
from jax.experimental import pallas as pl
from jax.experimental.pallas import tpu as pltpu
from jax.experimental.pallas import tpu_sc as plsc

B = 128
V = 100000
I32MAX = 2147483647

V_T = 55200
VC = 4600
NJT = V_T // VC
V_SC = V - V_T
NC = 2
NS = 16
NW = NC * NS
SW = V_SC // NW
CV = 200
NCHK = SW // CV
L = 16
NBG = B // L
NFLD = 6


def _recip(x):
    xi = plsc.bitcast(x, jnp.int32)
    y = plsc.bitcast(jnp.int32(0x7EF311C3) - xi, jnp.float32)
    for _ in range(4):
        y = y * (2.0 - x * y)
    return y


def _sc_body(ltf_hbm, temps_hbm, ntf_hbm, out_hbm,
             lb0, lb1, nb0, nb1, tbuf, state, seml0, seml1, semn0, semn1):
    c = lax.axis_index("c")
    s = lax.axis_index("s")
    wid = s * NC + c
    base = (V_T + wid * SW) * B

    pltpu.sync_copy(temps_hbm, tbuf)

    CHW = CV * B

    def issue(k, lb, nb, seml, semn):
        pltpu.async_copy(ltf_hbm.at[pl.ds(base + k * CHW, CHW)], lb, seml)
        pltpu.async_copy(ntf_hbm.at[pl.ds(base + k * CHW, CHW)], nb, semn)

    issue(0, lb0, nb0, seml0, semn0)
    issue(1, lb1, nb1, seml1, semn1)

    for bg in range(NBG):
        o = bg * (NFLD * L)
        state[pl.ds(o + 0 * L, L)] = jnp.full((L,), -jnp.inf, jnp.float32)
        state[pl.ds(o + 1 * L, L)] = jnp.zeros((L,), jnp.float32)
        state[pl.ds(o + 2 * L, L)] = jnp.ones((L,), jnp.float32)
        state[pl.ds(o + 3 * L, L)] = plsc.bitcast(
            jnp.zeros((L,), jnp.int32), jnp.float32)
        state[pl.ds(o + 4 * L, L)] = jnp.full((L,), -jnp.inf, jnp.float32)
        state[pl.ds(o + 5 * L, L)] = plsc.bitcast(
            jnp.zeros((L,), jnp.int32), jnp.float32)

    for k in range(NCHK):
        lb = (lb0, lb1)[k % 2]
        nb = (nb0, nb1)[k % 2]
        seml = (seml0, seml1)[k % 2]
        semn = (semn0, semn1)[k % 2]
        pltpu.make_async_copy(ltf_hbm.at[pl.ds(base, CHW)], lb, seml).wait()
        pltpu.make_async_copy(ntf_hbm.at[pl.ds(base, CHW)], nb, semn).wait()

        v0 = V_T + wid * SW + k * CV

        for bg in range(NBG):
            o = bg * (NFLD * L)
            c16 = bg * L
            t16 = tbuf[pl.ds(c16, L)]
            inv_t = _recip(jnp.where(t16 > 0.0, t16, 1.0))

            f_old = state[pl.ds(o + 0 * L, L)]
            b_e = state[pl.ds(o + 1 * L, L)]
            b_n = state[pl.ds(o + 2 * L, L)]
            bidx = plsc.bitcast(state[pl.ds(o + 3 * L, L)], jnp.int32)
            gmax = state[pl.ds(o + 4 * L, L)]
            gidx = plsc.bitcast(state[pl.ds(o + 5 * L, L)], jnp.int32)

            def pA(i, carry):
                cm, gm, gi = carry
                l = lb[pl.ds(i * B + c16, L)]
                upd = l > gm
                gi = jnp.where(upd, jnp.full((L,), v0, jnp.int32) + i, gi)
                gm = jnp.maximum(gm, l)
                cm = jnp.maximum(cm, l)
                return cm, gm, gi

            cm, gmax, gidx = lax.fori_loop(
                0, CV, pA,
                (jnp.full((L,), -jnp.inf, jnp.float32), gmax, gidx),
                unroll=4)

            f_new = jnp.maximum(f_old, cm * inv_t)
            b_e = b_e * jnp.exp(f_old - f_new)

            def pB(i, carry):
                b_e, b_n, bidx = carry
                l = lb[pl.ds(i * B + c16, L)]
                n = nb[pl.ds(i * B + c16, L)]
                e = jnp.exp(l * inv_t - f_new)
                ncl = jnp.maximum(n, 1e-10)
                upd = e * b_n > b_e * ncl
                b_e = jnp.where(upd, e, b_e)
                b_n = jnp.where(upd, ncl, b_n)
                bidx = jnp.where(upd, jnp.full((L,), v0, jnp.int32) + i, bidx)
                return b_e, b_n, bidx

            b_e, b_n, bidx = lax.fori_loop(0, CV, pB, (b_e, b_n, bidx),
                                           unroll=4)

            state[pl.ds(o + 0 * L, L)] = f_new
            state[pl.ds(o + 1 * L, L)] = b_e
            state[pl.ds(o + 2 * L, L)] = b_n
            state[pl.ds(o + 3 * L, L)] = plsc.bitcast(bidx, jnp.float32)
            state[pl.ds(o + 4 * L, L)] = gmax
            state[pl.ds(o + 5 * L, L)] = plsc.bitcast(gidx, jnp.float32)

        if k + 2 < NCHK:
            issue(k + 2, lb, nb, seml, semn)

    pltpu.sync_copy(state, out_hbm.at[wid])


def _sc_sampler(ltf, temperatures, ntf):
    mesh = plsc.VectorSubcoreMesh(core_axis_name="c", subcore_axis_name="s",
                                  num_cores=NC, num_subcores=NS)
    f = pl.kernel(
        _sc_body,
        out_type=jax.ShapeDtypeStruct((NW, NBG * NFLD * L), jnp.float32),
        mesh=mesh,
        scratch_types=[
            pltpu.VMEM((CV * B,), jnp.float32),
            pltpu.VMEM((CV * B,), jnp.float32),
            pltpu.VMEM((CV * B,), jnp.float32),
            pltpu.VMEM((CV * B,), jnp.float32),
            pltpu.VMEM((B,), jnp.float32),
            pltpu.VMEM((NBG * NFLD * L,), jnp.float32),
            pltpu.SemaphoreType.DMA,
            pltpu.SemaphoreType.DMA,
            pltpu.SemaphoreType.DMA,
            pltpu.SemaphoreType.DMA,
        ],
        compiler_params=pltpu.CompilerParams(needs_layout_passes=False),
    )
    return f(ltf, temperatures, ntf)


def _tc_body(lb, tb, nb, out_f, out_i, st):
    j = pl.program_id(0)

    t = tb[0, :].reshape(1, B)
    tpos = t > 0.0
    invt = 1.0 / jnp.where(tpos, t, 1.0)

    lblk = lb[...]
    nblk = nb[...]
    idx = j * VC + lax.broadcasted_iota(jnp.int32, (VC, B), 0)

    gm = jnp.max(lblk, axis=0, keepdims=True)
    mc = gm * invt
    e = jnp.exp((lblk - gm) * invt)
    ncl = jnp.maximum(nblk, 1e-10)
    r = e / ncl
    rm = jnp.max(r, axis=0, keepdims=True)
    ri = jnp.min(jnp.where(r == rm, idx, I32MAX), axis=0, keepdims=True)

    @pl.when(j == 0)
    def _():
        st[0:1, :] = mc
        st[1:2, :] = rm
        st[2:3, :] = lax.bitcast_convert_type(ri, jnp.float32)
        st[3:4, :] = gm
        st[4:5, :] = lax.bitcast_convert_type(
            jnp.zeros((1, B), jnp.int32), jnp.float32)

    @pl.when(jnp.any(~tpos))
    def _():
        gi = jnp.min(jnp.where(lblk == gm, idx, I32MAX), axis=0,
                     keepdims=True)
        gm_old = st[3:4, :]
        gi_old = lax.bitcast_convert_type(st[4:5, :], jnp.int32)
        first = jnp.logical_or(j == 0, gm > gm_old)
        st[4:5, :] = lax.bitcast_convert_type(
            jnp.where(first, gi, gi_old), jnp.float32)

    @pl.when(j > 0)
    def _():
        f_old = st[0:1, :]
        br = st[1:2, :]
        bi = lax.bitcast_convert_type(st[2:3, :], jnp.int32)
        f_new = jnp.maximum(f_old, mc)
        br = br * jnp.exp(f_old - f_new)
        cand = rm * jnp.exp(mc - f_new)
        upd = cand > br
        st[0:1, :] = f_new
        st[1:2, :] = jnp.where(upd, cand, br)
        st[2:3, :] = lax.bitcast_convert_type(jnp.where(upd, ri, bi),
                                              jnp.float32)
        st[3:4, :] = jnp.maximum(st[3:4, :], gm)

    @pl.when(j == NJT - 1)
    def _():
        out_f[0:1, :] = st[0:1, :]
        out_f[1:2, :] = st[1:2, :]
        out_f[2:3, :] = st[3:4, :]
        out_i[0:1, :] = lax.bitcast_convert_type(st[2:3, :], jnp.int32)
        out_i[1:2, :] = lax.bitcast_convert_type(st[4:5, :], jnp.int32)


def _tc_sampler(lt, temperatures, nt):
    f = pl.pallas_call(
        _tc_body,
        grid=(NJT,),
        in_specs=[
            pl.BlockSpec((VC, B), lambda j: (j, 0)),
            pl.BlockSpec((1, B), lambda j: (0, 0)),
            pl.BlockSpec((VC, B), lambda j: (j, 0)),
        ],
        out_specs=[
            pl.BlockSpec((8, B), lambda j: (0, 0)),
            pl.BlockSpec((8, B), lambda j: (0, 0)),
        ],
        out_shape=[
            jax.ShapeDtypeStruct((8, B), jnp.float32),
            jax.ShapeDtypeStruct((8, B), jnp.int32),
        ],
        scratch_shapes=[
            pltpu.VMEM((8, B), jnp.float32),
        ],
    )
    return f(lt, temperatures.reshape(1, B), nt)


@jax.jit
def _sampler(logits, temperatures, exp_noise):
    lt = logits.T
    nt = exp_noise.T
    sc_rec = _sc_sampler(lt.reshape(-1), temperatures, nt.reshape(-1))
    tc_f, tc_i = _tc_sampler(lt, temperatures, nt)

    rec = sc_rec.reshape(NW, NBG, NFLD, L)
    m_w = rec[:, :, 0, :].reshape(NW, B)
    e_w = rec[:, :, 1, :].reshape(NW, B)
    n_w = rec[:, :, 2, :].reshape(NW, B)
    i_w = lax.bitcast_convert_type(rec[:, :, 3, :], jnp.int32).reshape(NW, B)
    gm_w = rec[:, :, 4, :].reshape(NW, B)
    gi_w = lax.bitcast_convert_type(rec[:, :, 5, :], jnp.int32).reshape(NW, B)

    f_t, r_t, gm_t = tc_f[0], tc_f[1], tc_f[2]
    i_t, gi_t = tc_i[0], tc_i[1]

    m_all = jnp.concatenate([f_t[None], m_w], axis=0)
    key_all = jnp.concatenate([r_t[None], e_w / n_w], axis=0)
    idx_all = jnp.concatenate([i_t[None], i_w], axis=0)
    mg = jnp.max(m_all, axis=0)
    keys = key_all * jnp.exp(m_all - mg)
    km = jnp.max(keys, axis=0)
    stok = jnp.min(jnp.where(keys == km, idx_all, I32MAX), axis=0)

    gm_all = jnp.concatenate([gm_t[None], gm_w], axis=0)
    gi_all = jnp.concatenate([gi_t[None], gi_w], axis=0)
    g = jnp.max(gm_all, axis=0)
    gtok = jnp.min(jnp.where(gm_all == g, gi_all, I32MAX), axis=0)

    return jnp.where(temperatures > 0.0, stok, gtok)


def kernel(logits, temperatures, exp_noise):
    return _sampler(logits, temperatures, exp_noise)


# speedup vs baseline: 4.5295x; 1.1408x over previous
import functools

import jax
import jax.numpy as jnp
from jax import lax
from jax.experimental import pallas as pl
from jax.experimental.pallas import tpu as pltpu
from jax.experimental.pallas import tpu_sc as plsc

B = 128
V = 100000
I32MAX = 2147483647

V_T = 66720
VC = 11120
NJT = V_T // VC
V_SC = V - V_T
NC = 2
NS = 16
NW = NC * NS
SW = V_SC // NW
CV = 208
NCHK = SW // CV
L = 16
NBG = B // L
NFLD = 6


def _recip(x):
    xi = plsc.bitcast(x, jnp.int32)
    y = plsc.bitcast(jnp.int32(0x7EF311C3) - xi, jnp.float32)
    for _ in range(4):
        y = y * (2.0 - x * y)
    return y


def _sc_body(ltf_hbm, temps_hbm, ntf_hbm, out_hbm,
             lb0, lb1, nb0, nb1, tbuf, state, seml0, seml1, semn0, semn1):
    c = lax.axis_index("c")
    s = lax.axis_index("s")
    wid = s * NC + c
    base = (V_T + wid * SW) * B

    pltpu.sync_copy(temps_hbm, tbuf)

    CHW = CV * B

    def issue(k, lb, nb, seml, semn):
        pltpu.async_copy(ltf_hbm.at[pl.ds(base + k * CHW, CHW)], lb, seml)
        pltpu.async_copy(ntf_hbm.at[pl.ds(base + k * CHW, CHW)], nb, semn)

    issue(0, lb0, nb0, seml0, semn0)
    issue(1, lb1, nb1, seml1, semn1)

    for bg in range(NBG):
        o = bg * (NFLD * L)
        state[pl.ds(o + 0 * L, L)] = jnp.full((L,), -jnp.inf, jnp.float32)
        state[pl.ds(o + 1 * L, L)] = jnp.zeros((L,), jnp.float32)
        state[pl.ds(o + 2 * L, L)] = jnp.ones((L,), jnp.float32)
        state[pl.ds(o + 3 * L, L)] = plsc.bitcast(
            jnp.zeros((L,), jnp.int32), jnp.float32)
        state[pl.ds(o + 4 * L, L)] = jnp.full((L,), -jnp.inf, jnp.float32)
        state[pl.ds(o + 5 * L, L)] = plsc.bitcast(
            jnp.zeros((L,), jnp.int32), jnp.float32)

    for k in range(NCHK):
        lb = (lb0, lb1)[k % 2]
        nb = (nb0, nb1)[k % 2]
        seml = (seml0, seml1)[k % 2]
        semn = (semn0, semn1)[k % 2]
        pltpu.make_async_copy(ltf_hbm.at[pl.ds(base, CHW)], lb, seml).wait()
        pltpu.make_async_copy(ntf_hbm.at[pl.ds(base, CHW)], nb, semn).wait()

        v0 = V_T + wid * SW + k * CV

        for bg in range(NBG):
            o = bg * (NFLD * L)
            c16 = bg * L
            t16 = tbuf[pl.ds(c16, L)]
            inv_t = _recip(jnp.where(t16 > 0.0, t16, 1.0))

            f_old = state[pl.ds(o + 0 * L, L)]
            b_e = state[pl.ds(o + 1 * L, L)]
            b_n = state[pl.ds(o + 2 * L, L)]
            bidx = plsc.bitcast(state[pl.ds(o + 3 * L, L)], jnp.int32)
            gmax = state[pl.ds(o + 4 * L, L)]
            gidx = plsc.bitcast(state[pl.ds(o + 5 * L, L)], jnp.int32)

            def pA(i, carry):
                cm, gm, gi = carry
                l = lb[pl.ds(i * B + c16, L)]
                upd = l > gm
                gi = jnp.where(upd, jnp.full((L,), v0, jnp.int32) + i, gi)
                gm = jnp.maximum(gm, l)
                cm = jnp.maximum(cm, l)
                return cm, gm, gi

            cm, gmax, gidx = lax.fori_loop(
                0, CV, pA,
                (jnp.full((L,), -jnp.inf, jnp.float32), gmax, gidx),
                unroll=4)

            f_new = jnp.maximum(f_old, cm * inv_t)
            b_e = b_e * jnp.exp(f_old - f_new)

            def pB(i, carry):
                b_e, b_n, bidx = carry
                l = lb[pl.ds(i * B + c16, L)]
                n = nb[pl.ds(i * B + c16, L)]
                e = jnp.exp(l * inv_t - f_new)
                ncl = jnp.maximum(n, 1e-10)
                upd = e * b_n > b_e * ncl
                b_e = jnp.where(upd, e, b_e)
                b_n = jnp.where(upd, ncl, b_n)
                bidx = jnp.where(upd, jnp.full((L,), v0, jnp.int32) + i, bidx)
                return b_e, b_n, bidx

            b_e, b_n, bidx = lax.fori_loop(0, CV, pB, (b_e, b_n, bidx),
                                           unroll=4)

            state[pl.ds(o + 0 * L, L)] = f_new
            state[pl.ds(o + 1 * L, L)] = b_e
            state[pl.ds(o + 2 * L, L)] = b_n
            state[pl.ds(o + 3 * L, L)] = plsc.bitcast(bidx, jnp.float32)
            state[pl.ds(o + 4 * L, L)] = gmax
            state[pl.ds(o + 5 * L, L)] = plsc.bitcast(gidx, jnp.float32)

        if k + 2 < NCHK:
            issue(k + 2, lb, nb, seml, semn)

    pltpu.sync_copy(state, out_hbm.at[wid])


def _sc_sampler(ltf, temperatures, ntf):
    mesh = plsc.VectorSubcoreMesh(core_axis_name="c", subcore_axis_name="s",
                                  num_cores=NC, num_subcores=NS)
    f = pl.kernel(
        _sc_body,
        out_type=jax.ShapeDtypeStruct((NW, NBG * NFLD * L), jnp.float32),
        mesh=mesh,
        scratch_types=[
            pltpu.VMEM((CV * B,), jnp.float32),
            pltpu.VMEM((CV * B,), jnp.float32),
            pltpu.VMEM((CV * B,), jnp.float32),
            pltpu.VMEM((CV * B,), jnp.float32),
            pltpu.VMEM((B,), jnp.float32),
            pltpu.VMEM((NBG * NFLD * L,), jnp.float32),
            pltpu.SemaphoreType.DMA,
            pltpu.SemaphoreType.DMA,
            pltpu.SemaphoreType.DMA,
            pltpu.SemaphoreType.DMA,
        ],
        compiler_params=pltpu.CompilerParams(needs_layout_passes=False),
    )
    return f(ltf, temperatures, ntf)


def _tc_body(lb, tb, nb, out_f, out_i, st):
    j = pl.program_id(0)

    t = tb[0, :].reshape(1, B)
    tpos = t > 0.0
    invt = 1.0 / jnp.where(tpos, t, 1.0)

    lblk = lb[...]
    nblk = nb[...]
    idx = j * VC + lax.broadcasted_iota(jnp.int32, (VC, B), 0)

    gm = jnp.max(lblk, axis=0, keepdims=True)
    mc = gm * invt
    e = jnp.exp((lblk - gm) * invt)
    ncl = jnp.maximum(nblk, 1e-10)
    r = e / ncl
    rm = jnp.max(r, axis=0, keepdims=True)
    ri = jnp.min(jnp.where(r == rm, idx, I32MAX), axis=0, keepdims=True)

    @pl.when(j == 0)
    def _():
        st[0:1, :] = mc
        st[1:2, :] = rm
        st[2:3, :] = lax.bitcast_convert_type(ri, jnp.float32)
        st[3:4, :] = gm
        st[4:5, :] = lax.bitcast_convert_type(
            jnp.zeros((1, B), jnp.int32), jnp.float32)

    @pl.when(jnp.any(~tpos))
    def _():
        gi = jnp.min(jnp.where(lblk == gm, idx, I32MAX), axis=0,
                     keepdims=True)
        gm_old = st[3:4, :]
        gi_old = lax.bitcast_convert_type(st[4:5, :], jnp.int32)
        first = jnp.logical_or(j == 0, gm > gm_old)
        st[4:5, :] = lax.bitcast_convert_type(
            jnp.where(first, gi, gi_old), jnp.float32)

    @pl.when(j > 0)
    def _():
        f_old = st[0:1, :]
        br = st[1:2, :]
        bi = lax.bitcast_convert_type(st[2:3, :], jnp.int32)
        f_new = jnp.maximum(f_old, mc)
        br = br * jnp.exp(f_old - f_new)
        cand = rm * jnp.exp(mc - f_new)
        upd = cand > br
        st[0:1, :] = f_new
        st[1:2, :] = jnp.where(upd, cand, br)
        st[2:3, :] = lax.bitcast_convert_type(jnp.where(upd, ri, bi),
                                              jnp.float32)
        st[3:4, :] = jnp.maximum(st[3:4, :], gm)

    @pl.when(j == NJT - 1)
    def _():
        out_f[0:1, :] = st[0:1, :]
        out_f[1:2, :] = st[1:2, :]
        out_f[2:3, :] = st[3:4, :]
        out_i[0:1, :] = lax.bitcast_convert_type(st[2:3, :], jnp.int32)
        out_i[1:2, :] = lax.bitcast_convert_type(st[4:5, :], jnp.int32)


def _tc_sampler(lt, temperatures, nt):
    f = pl.pallas_call(
        _tc_body,
        grid=(NJT,),
        in_specs=[
            pl.BlockSpec((VC, B), lambda j: (j, 0)),
            pl.BlockSpec((1, B), lambda j: (0, 0)),
            pl.BlockSpec((VC, B), lambda j: (j, 0)),
        ],
        out_specs=[
            pl.BlockSpec((8, B), lambda j: (0, 0)),
            pl.BlockSpec((8, B), lambda j: (0, 0)),
        ],
        out_shape=[
            jax.ShapeDtypeStruct((8, B), jnp.float32),
            jax.ShapeDtypeStruct((8, B), jnp.int32),
        ],
        scratch_shapes=[
            pltpu.VMEM((8, B), jnp.float32),
        ],
    )
    return f(lt, temperatures.reshape(1, B), nt)


@jax.jit
def _sampler(logits, temperatures, exp_noise):
    lt = logits.T
    nt = exp_noise.T
    sc_rec = _sc_sampler(lt.reshape(-1), temperatures, nt.reshape(-1))
    tc_f, tc_i = _tc_sampler(lt, temperatures, nt)

    rec = sc_rec.reshape(NW, NBG, NFLD, L)
    m_w = rec[:, :, 0, :].reshape(NW, B)
    e_w = rec[:, :, 1, :].reshape(NW, B)
    n_w = rec[:, :, 2, :].reshape(NW, B)
    i_w = lax.bitcast_convert_type(rec[:, :, 3, :], jnp.int32).reshape(NW, B)
    gm_w = rec[:, :, 4, :].reshape(NW, B)
    gi_w = lax.bitcast_convert_type(rec[:, :, 5, :], jnp.int32).reshape(NW, B)

    f_t, r_t, gm_t = tc_f[0], tc_f[1], tc_f[2]
    i_t, gi_t = tc_i[0], tc_i[1]

    m_all = jnp.concatenate([f_t[None], m_w], axis=0)
    key_all = jnp.concatenate([r_t[None], e_w / n_w], axis=0)
    idx_all = jnp.concatenate([i_t[None], i_w], axis=0)
    mg = jnp.max(m_all, axis=0)
    keys = key_all * jnp.exp(m_all - mg)
    km = jnp.max(keys, axis=0)
    stok = jnp.min(jnp.where(keys == km, idx_all, I32MAX), axis=0)

    gm_all = jnp.concatenate([gm_t[None], gm_w], axis=0)
    gi_all = jnp.concatenate([gi_t[None], gi_w], axis=0)
    g = jnp.max(gm_all, axis=0)
    gtok = jnp.min(jnp.where(gm_all == g, gi_all, I32MAX), axis=0)

    return jnp.where(temperatures > 0.0, stok, gtok)


def kernel(logits, temperatures, exp_noise):
    return _sampler(logits, temperatures, exp_noise)


# speedup vs baseline: 4.9285x; 1.0881x over previous
import functools

import jax
import jax.numpy as jnp
from jax import lax
from jax.experimental import pallas as pl
from jax.experimental.pallas import tpu as pltpu
from jax.experimental.pallas import tpu_sc as plsc

B = 128
V = 100000
I32MAX = 2147483647

V_T = 72352
VC = 10336
NJT = V_T // VC
V_SC = V - V_T
NC = 2
NS = 16
NW = NC * NS
SW = V_SC // NW
CV = 216
NCHK = SW // CV
L = 16
NBG = B // L
NFLD = 6


def _recip(x):
    xi = plsc.bitcast(x, jnp.int32)
    y = plsc.bitcast(jnp.int32(0x7EF311C3) - xi, jnp.float32)
    for _ in range(4):
        y = y * (2.0 - x * y)
    return y


def _sc_body(ltf_hbm, temps_hbm, ntf_hbm, out_hbm,
             lb0, lb1, nb0, nb1, tbuf, state, seml0, seml1, semn0, semn1):
    c = lax.axis_index("c")
    s = lax.axis_index("s")
    wid = s * NC + c
    base = (V_T + wid * SW) * B

    pltpu.sync_copy(temps_hbm, tbuf)

    CHW = CV * B

    def issue(k, lb, nb, seml, semn):
        pltpu.async_copy(ltf_hbm.at[pl.ds(base + k * CHW, CHW)], lb, seml)
        pltpu.async_copy(ntf_hbm.at[pl.ds(base + k * CHW, CHW)], nb, semn)

    issue(0, lb0, nb0, seml0, semn0)
    issue(1, lb1, nb1, seml1, semn1)

    for bg in range(NBG):
        o = bg * (NFLD * L)
        state[pl.ds(o + 0 * L, L)] = jnp.full((L,), -jnp.inf, jnp.float32)
        state[pl.ds(o + 1 * L, L)] = jnp.zeros((L,), jnp.float32)
        state[pl.ds(o + 2 * L, L)] = jnp.ones((L,), jnp.float32)
        state[pl.ds(o + 3 * L, L)] = plsc.bitcast(
            jnp.zeros((L,), jnp.int32), jnp.float32)
        state[pl.ds(o + 4 * L, L)] = jnp.full((L,), -jnp.inf, jnp.float32)
        state[pl.ds(o + 5 * L, L)] = plsc.bitcast(
            jnp.zeros((L,), jnp.int32), jnp.float32)

    for k in range(NCHK):
        lb = (lb0, lb1)[k % 2]
        nb = (nb0, nb1)[k % 2]
        seml = (seml0, seml1)[k % 2]
        semn = (semn0, semn1)[k % 2]
        pltpu.make_async_copy(ltf_hbm.at[pl.ds(base, CHW)], lb, seml).wait()
        pltpu.make_async_copy(ntf_hbm.at[pl.ds(base, CHW)], nb, semn).wait()

        v0 = V_T + wid * SW + k * CV

        for bg in range(NBG):
            o = bg * (NFLD * L)
            c16 = bg * L
            t16 = tbuf[pl.ds(c16, L)]
            inv_t = _recip(jnp.where(t16 > 0.0, t16, 1.0))

            f_old = state[pl.ds(o + 0 * L, L)]
            b_e = state[pl.ds(o + 1 * L, L)]
            b_n = state[pl.ds(o + 2 * L, L)]
            bidx = plsc.bitcast(state[pl.ds(o + 3 * L, L)], jnp.int32)
            gmax = state[pl.ds(o + 4 * L, L)]
            gidx = plsc.bitcast(state[pl.ds(o + 5 * L, L)], jnp.int32)

            def pA(i, carry):
                cm, gm, gi = carry
                l = lb[pl.ds(i * B + c16, L)]
                upd = l > gm
                gi = jnp.where(upd, jnp.full((L,), v0, jnp.int32) + i, gi)
                gm = jnp.maximum(gm, l)
                cm = jnp.maximum(cm, l)
                return cm, gm, gi

            cm, gmax, gidx = lax.fori_loop(
                0, CV, pA,
                (jnp.full((L,), -jnp.inf, jnp.float32), gmax, gidx),
                unroll=4)

            f_new = jnp.maximum(f_old, cm * inv_t)
            b_e = b_e * jnp.exp(f_old - f_new)

            def pB(i, carry):
                b_e, b_n, bidx = carry
                l = lb[pl.ds(i * B + c16, L)]
                n = nb[pl.ds(i * B + c16, L)]
                e = jnp.exp(l * inv_t - f_new)
                ncl = jnp.maximum(n, 1e-10)
                upd = e * b_n > b_e * ncl
                b_e = jnp.where(upd, e, b_e)
                b_n = jnp.where(upd, ncl, b_n)
                bidx = jnp.where(upd, jnp.full((L,), v0, jnp.int32) + i, bidx)
                return b_e, b_n, bidx

            b_e, b_n, bidx = lax.fori_loop(0, CV, pB, (b_e, b_n, bidx),
                                           unroll=4)

            state[pl.ds(o + 0 * L, L)] = f_new
            state[pl.ds(o + 1 * L, L)] = b_e
            state[pl.ds(o + 2 * L, L)] = b_n
            state[pl.ds(o + 3 * L, L)] = plsc.bitcast(bidx, jnp.float32)
            state[pl.ds(o + 4 * L, L)] = gmax
            state[pl.ds(o + 5 * L, L)] = plsc.bitcast(gidx, jnp.float32)

        if k + 2 < NCHK:
            issue(k + 2, lb, nb, seml, semn)

    pltpu.sync_copy(state, out_hbm.at[wid])


def _sc_sampler(ltf, temperatures, ntf):
    mesh = plsc.VectorSubcoreMesh(core_axis_name="c", subcore_axis_name="s",
                                  num_cores=NC, num_subcores=NS)
    f = pl.kernel(
        _sc_body,
        out_type=jax.ShapeDtypeStruct((NW, NBG * NFLD * L), jnp.float32),
        mesh=mesh,
        scratch_types=[
            pltpu.VMEM((CV * B,), jnp.float32),
            pltpu.VMEM((CV * B,), jnp.float32),
            pltpu.VMEM((CV * B,), jnp.float32),
            pltpu.VMEM((CV * B,), jnp.float32),
            pltpu.VMEM((B,), jnp.float32),
            pltpu.VMEM((NBG * NFLD * L,), jnp.float32),
            pltpu.SemaphoreType.DMA,
            pltpu.SemaphoreType.DMA,
            pltpu.SemaphoreType.DMA,
            pltpu.SemaphoreType.DMA,
        ],
        compiler_params=pltpu.CompilerParams(needs_layout_passes=False),
    )
    return f(ltf, temperatures, ntf)


def _tc_body(lb, tb, nb, out_f, out_i, st):
    j = pl.program_id(0)

    t = tb[0, :].reshape(1, B)
    tpos = t > 0.0
    invt = 1.0 / jnp.where(tpos, t, 1.0)

    lblk = lb[...]
    nblk = nb[...]
    idx = j * VC + lax.broadcasted_iota(jnp.int32, (VC, B), 0)

    gm = jnp.max(lblk, axis=0, keepdims=True)
    mc = gm * invt
    e = jnp.exp((lblk - gm) * invt)
    ncl = jnp.maximum(nblk, 1e-10)
    r = e / ncl
    rm = jnp.max(r, axis=0, keepdims=True)
    ri = jnp.min(jnp.where(r == rm, idx, I32MAX), axis=0, keepdims=True)

    @pl.when(j == 0)
    def _():
        st[0:1, :] = mc
        st[1:2, :] = rm
        st[2:3, :] = lax.bitcast_convert_type(ri, jnp.float32)
        st[3:4, :] = gm
        st[4:5, :] = lax.bitcast_convert_type(
            jnp.zeros((1, B), jnp.int32), jnp.float32)

    @pl.when(jnp.any(~tpos))
    def _():
        gi = jnp.min(jnp.where(lblk == gm, idx, I32MAX), axis=0,
                     keepdims=True)
        gm_old = st[3:4, :]
        gi_old = lax.bitcast_convert_type(st[4:5, :], jnp.int32)
        first = jnp.logical_or(j == 0, gm > gm_old)
        st[4:5, :] = lax.bitcast_convert_type(
            jnp.where(first, gi, gi_old), jnp.float32)

    @pl.when(j > 0)
    def _():
        f_old = st[0:1, :]
        br = st[1:2, :]
        bi = lax.bitcast_convert_type(st[2:3, :], jnp.int32)
        f_new = jnp.maximum(f_old, mc)
        br = br * jnp.exp(f_old - f_new)
        cand = rm * jnp.exp(mc - f_new)
        upd = cand > br
        st[0:1, :] = f_new
        st[1:2, :] = jnp.where(upd, cand, br)
        st[2:3, :] = lax.bitcast_convert_type(jnp.where(upd, ri, bi),
                                              jnp.float32)
        st[3:4, :] = jnp.maximum(st[3:4, :], gm)

    @pl.when(j == NJT - 1)
    def _():
        out_f[0:1, :] = st[0:1, :]
        out_f[1:2, :] = st[1:2, :]
        out_f[2:3, :] = st[3:4, :]
        out_i[0:1, :] = lax.bitcast_convert_type(st[2:3, :], jnp.int32)
        out_i[1:2, :] = lax.bitcast_convert_type(st[4:5, :], jnp.int32)


def _tc_sampler(lt, temperatures, nt):
    f = pl.pallas_call(
        _tc_body,
        grid=(NJT,),
        in_specs=[
            pl.BlockSpec((VC, B), lambda j: (j, 0)),
            pl.BlockSpec((1, B), lambda j: (0, 0)),
            pl.BlockSpec((VC, B), lambda j: (j, 0)),
        ],
        out_specs=[
            pl.BlockSpec((8, B), lambda j: (0, 0)),
            pl.BlockSpec((8, B), lambda j: (0, 0)),
        ],
        out_shape=[
            jax.ShapeDtypeStruct((8, B), jnp.float32),
            jax.ShapeDtypeStruct((8, B), jnp.int32),
        ],
        scratch_shapes=[
            pltpu.VMEM((8, B), jnp.float32),
        ],
    )
    return f(lt, temperatures.reshape(1, B), nt)


@jax.jit
def _sampler(logits, temperatures, exp_noise):
    lt = logits.T
    nt = exp_noise.T
    sc_rec = _sc_sampler(lt.reshape(-1), temperatures, nt.reshape(-1))
    tc_f, tc_i = _tc_sampler(lt, temperatures, nt)

    rec = sc_rec.reshape(NW, NBG, NFLD, L)
    m_w = rec[:, :, 0, :].reshape(NW, B)
    e_w = rec[:, :, 1, :].reshape(NW, B)
    n_w = rec[:, :, 2, :].reshape(NW, B)
    i_w = lax.bitcast_convert_type(rec[:, :, 3, :], jnp.int32).reshape(NW, B)
    gm_w = rec[:, :, 4, :].reshape(NW, B)
    gi_w = lax.bitcast_convert_type(rec[:, :, 5, :], jnp.int32).reshape(NW, B)

    f_t, r_t, gm_t = tc_f[0], tc_f[1], tc_f[2]
    i_t, gi_t = tc_i[0], tc_i[1]

    m_all = jnp.concatenate([f_t[None], m_w], axis=0)
    key_all = jnp.concatenate([r_t[None], e_w / n_w], axis=0)
    idx_all = jnp.concatenate([i_t[None], i_w], axis=0)
    mg = jnp.max(m_all, axis=0)
    keys = key_all * jnp.exp(m_all - mg)
    km = jnp.max(keys, axis=0)
    stok = jnp.min(jnp.where(keys == km, idx_all, I32MAX), axis=0)

    gm_all = jnp.concatenate([gm_t[None], gm_w], axis=0)
    gi_all = jnp.concatenate([gi_t[None], gi_w], axis=0)
    g = jnp.max(gm_all, axis=0)
    gtok = jnp.min(jnp.where(gm_all == g, gi_all, I32MAX), axis=0)

    return jnp.where(temperatures > 0.0, stok, gtok)


def kernel(logits, temperatures, exp_noise):
    return _sampler(logits, temperatures, exp_noise)
